# baseline (speedup 1.0000x reference)
import functools

import jax
import jax.numpy as jnp
from jax import lax
from jax.experimental import pallas as pl
from jax.experimental.pallas import tpu as pltpu
from jax.experimental.pallas import tpu_sc as plsc

NC = 2
NS = 16
LN = 16
CHW = 128


DW = 128


def _sc_degree(num_chunks: int, np_: int):
    slc = np_ // NS
    mesh = plsc.VectorSubcoreMesh(core_axis_name="c", subcore_axis_name="s")

    @functools.partial(
        pl.kernel,
        out_type=jax.ShapeDtypeStruct((NC * np_, DW), jnp.float32),
        mesh=mesh,
        scratch_types=[
            pltpu.VMEM((num_chunks, CHW), jnp.int32),
            pltpu.VMEM((CHW, DW), jnp.float32),
            pltpu.VMEM_SHARED((np_, DW), jnp.float32),
        ],
    )
    def deg_kernel(didx_hbm, out_hbm, idx_v, ones_v, hist_sh):
        c = lax.axis_index("c")
        s = lax.axis_index("s")
        w = c * NS + s
        pltpu.sync_copy(didx_hbm.at[w], idx_v)
        zv = jnp.zeros((LN,), jnp.float32)

        @pl.loop(0, CHW)
        def _(r):
            @pl.loop(0, DW, step=LN)
            def _(k):
                ones_v[r, pl.ds(k, LN)] = zv

        @pl.loop(0, slc // CHW)
        def _(k):
            pltpu.sync_copy(ones_v, hist_sh.at[pl.ds(s * slc + k * CHW, CHW)])

        lane = lax.iota(jnp.int32, LN)
        e1 = jnp.where(lane == 0, 1.0, 0.0).astype(jnp.float32)

        @pl.loop(0, CHW)
        def _(r):
            ones_v[r, pl.ds(0, LN)] = e1

        plsc.subcore_barrier()

        @pl.loop(0, num_chunks)
        def _(j):
            pltpu.sync_copy(ones_v, hist_sh.at[idx_v.at[j]], add=True)

        plsc.subcore_barrier()
        pltpu.sync_copy(
            hist_sh.at[pl.ds(s * slc, slc)],
            out_hbm.at[pl.ds(c * np_ + s * slc, slc)],
        )

    return deg_kernel


WIN = 32
CW = 64


def _sc_scatter(num_chunks: int, np_: int, dh: int):
    slc = np_ // NS
    grp = num_chunks // WIN
    mesh = plsc.VectorSubcoreMesh(core_axis_name="c", subcore_axis_name="s")

    @functools.partial(
        pl.kernel,
        out_type=jax.ShapeDtypeStruct((NC * np_, dh), jnp.float32),
        mesh=mesh,
        scratch_types=[
            pltpu.VMEM((WIN, CW), jnp.int32),
            pltpu.VMEM((WIN, CW), jnp.int32),
            [pltpu.VMEM((CW, dh), jnp.float32) for _ in range(4)],
            [pltpu.SemaphoreType.DMA for _ in range(8)],
            pltpu.VMEM_SHARED((np_, dh), jnp.float32),
        ],
    )
    def scat_kernel(hs_hbm, sidx_hbm, didx_hbm, out_hbm,
                    si_v, di_v, bufs, sems, acc_sh):
        c = lax.axis_index("c")
        s = lax.axis_index("s")
        w = c * NS + s
        sg = sems[:4]
        ss = sems[4:]
        zv = jnp.zeros((LN,), jnp.float32)

        @pl.loop(0, CW)
        def _(r):
            @pl.loop(0, dh, step=LN)
            def _(k):
                bufs[0][r, pl.ds(k, LN)] = zv

        @pl.loop(0, slc // CW)
        def _(k):
            pltpu.sync_copy(bufs[0], acc_sh.at[pl.ds(s * slc + k * CW, CW)])

        plsc.subcore_barrier()

        def gather(j, i):
            pltpu.async_copy(hs_hbm.at[si_v.at[j]], bufs[i], sg[i])

        def gather_wait(j, i):
            pltpu.make_async_copy(hs_hbm.at[si_v.at[j]], bufs[i],
                                  sg[i]).wait()

        def scat(j, i):
            pltpu.async_copy(bufs[i], acc_sh.at[di_v.at[j]], ss[i], add=True)

        def scat_wait(j, i):
            pltpu.make_async_copy(bufs[i], acc_sh.at[di_v.at[j]],
                                  ss[i]).wait()

        @pl.loop(0, grp)
        def _(g):
            pltpu.sync_copy(sidx_hbm.at[w * grp + g], si_v)
            pltpu.sync_copy(didx_hbm.at[s * grp + g], di_v)
            gather(0, 0)
            gather(1, 1)

            @pl.loop(0, WIN, step=4)
            def _(j):
                for i in range(4):
                    jj = j + i
                    gather_wait(jj, i)
                    scat(jj, i)
                    i2 = (i + 2) % 4

                    @pl.when(jj + 2 < WIN)
                    def _(jj=jj, i2=i2):
                        @pl.when(jj >= 2)
                        def _():
                            scat_wait(jj - 2, i2)

                        gather(jj + 2, i2)

            for i in range(4):
                scat_wait(WIN - 4 + i, i)

        plsc.subcore_barrier()
        pltpu.sync_copy(
            acc_sh.at[pl.ds(s * slc, slc)],
            out_hbm.at[pl.ds(c * np_ + s * slc, slc)],
        )

    return scat_kernel


def _tc_matmul(np_: int, blk: int, din: int, dh: int):
    nb = np_ // blk

    def body(x_ref, w_ref, da_ref, db_ref, o_ref):
        deg = da_ref[:, 0] + db_ref[:, 0] + 1.0
        dinv = lax.rsqrt(deg)
        xs = x_ref[...] * dinv[:, None]
        o_ref[...] = jnp.dot(xs, w_ref[...],
                             preferred_element_type=jnp.float32)

    return pl.pallas_call(
        body,
        grid=(nb, 2),
        in_specs=[
            pl.BlockSpec((blk, din), lambda i, j: (i, 0)),
            pl.BlockSpec((din, dh), lambda i, j: (0, j)),
            pl.BlockSpec((blk, DW), lambda i, j: (i, 0)),
            pl.BlockSpec((blk, DW), lambda i, j: (nb + i, 0)),
        ],
        out_specs=pl.BlockSpec((blk, dh), lambda i, j: (j * nb + i, 0)),
        out_shape=jax.ShapeDtypeStruct((NC * np_, dh), jnp.float32),
    )


def _tc_final(np_: int, blk: int, dh: int):
    nb = np_ // blk

    def body(aa_ref, ab_ref, ha_ref, hb_ref, da_ref, db_ref, b_ref, o_ref):
        deg = da_ref[:, 0] + db_ref[:, 0] + 1.0
        dinv = lax.rsqrt(deg)[:, None]
        left = dinv * (aa_ref[...] + ha_ref[...])
        right = dinv * (ab_ref[...] + hb_ref[...])
        o_ref[...] = jnp.maximum(
            jnp.concatenate([left, right], axis=1) + b_ref[...], 0.0)

    return pl.pallas_call(
        body,
        grid=(nb,),
        in_specs=[
            pl.BlockSpec((blk, dh), lambda i: (i, 0)),
            pl.BlockSpec((blk, dh), lambda i: (nb + i, 0)),
            pl.BlockSpec((blk, dh), lambda i: (i, 0)),
            pl.BlockSpec((blk, dh), lambda i: (nb + i, 0)),
            pl.BlockSpec((blk, DW), lambda i: (i, 0)),
            pl.BlockSpec((blk, DW), lambda i: (nb + i, 0)),
            pl.BlockSpec((1, 2 * dh), lambda i: (0, 0)),
        ],
        out_specs=pl.BlockSpec((blk, 2 * dh), lambda i: (i, 0)),
        out_shape=jax.ShapeDtypeStruct((np_, 2 * dh), jnp.float32),
    )


def kernel(x, spatial_edge_index, W, b):
    n, din = x.shape
    dout = W.shape[1]
    dh = dout // 2
    e = spatial_edge_index.shape[1]

    blk = 1024
    tile_q = NS * CW * WIN
    np_ = ((n + 1 + 2047) // 2048) * 2048
    np_ = ((np_ + blk - 1) // blk) * blk
    ep = ((e + tile_q - 1) // tile_q) * tile_q

    src = spatial_edge_index[0].astype(jnp.int32)
    dst = spatial_edge_index[1].astype(jnp.int32)
    pad = ep - e
    srcp = jnp.concatenate([src, jnp.full((pad,), n, jnp.int32)])
    dstp = jnp.concatenate([dst, jnp.full((pad,), n, jnp.int32)])

    ch_m = ep // (NS * CW)
    ch_d = ep // (NC * NS * CHW)
    grp = ch_m // WIN

    src_by_s = srcp.reshape(NS, ch_m, CW)
    src_arr = jnp.concatenate([src_by_s, src_by_s + np_],
                              axis=0).reshape(NC * NS * grp, WIN, CW)
    dst_arr = dstp.reshape(NS * grp, WIN, CW)
    deg_idx = dstp.reshape(NC * NS, ch_d, CHW)

    x_pad = jnp.concatenate(
        [x, jnp.zeros((np_ - n, din), x.dtype)], axis=0)

    deg01 = _sc_degree(ch_d, np_)(deg_idx)
    hs = _tc_matmul(np_, blk, din, dh)(x_pad, W, deg01, deg01)
    acc = _sc_scatter(ch_m, np_, dh)(hs, src_arr, dst_arr)
    out_p = _tc_final(np_, blk, dh)(acc, acc, hs, hs, deg01, deg01,
                                    b.reshape(1, dout))
    return out_p[:n]


# speedup vs baseline: 1.0510x; 1.0510x over previous
import functools

import jax
import jax.numpy as jnp
from jax import lax
from jax.experimental import pallas as pl
from jax.experimental.pallas import tpu as pltpu
from jax.experimental.pallas import tpu_sc as plsc

NC = 2
NS = 16
LN = 16
CHW = 128


DW = 128


def _sc_degree(num_chunks: int, np_: int):
    slc = np_ // NS
    mesh = plsc.VectorSubcoreMesh(core_axis_name="c", subcore_axis_name="s")

    @functools.partial(
        pl.kernel,
        out_type=jax.ShapeDtypeStruct((NC * np_, DW), jnp.float32),
        mesh=mesh,
        scratch_types=[
            pltpu.VMEM((num_chunks, CHW), jnp.int32),
            pltpu.VMEM((CHW, DW), jnp.float32),
            pltpu.VMEM_SHARED((np_, DW), jnp.float32),
        ],
    )
    def deg_kernel(didx_hbm, out_hbm, idx_v, ones_v, hist_sh):
        c = lax.axis_index("c")
        s = lax.axis_index("s")
        w = c * NS + s
        pltpu.sync_copy(didx_hbm.at[w], idx_v)
        zv = jnp.zeros((LN,), jnp.float32)

        @pl.loop(0, CHW)
        def _(r):
            @pl.loop(0, DW, step=LN)
            def _(k):
                ones_v[r, pl.ds(k, LN)] = zv

        @pl.loop(0, slc // CHW)
        def _(k):
            pltpu.sync_copy(ones_v, hist_sh.at[pl.ds(s * slc + k * CHW, CHW)])

        lane = lax.iota(jnp.int32, LN)
        e1 = jnp.where(lane == 0, 1.0, 0.0).astype(jnp.float32)

        @pl.loop(0, CHW)
        def _(r):
            ones_v[r, pl.ds(0, LN)] = e1

        plsc.subcore_barrier()

        @pl.loop(0, num_chunks)
        def _(j):
            pltpu.sync_copy(ones_v, hist_sh.at[idx_v.at[j]], add=True)

        plsc.subcore_barrier()
        pltpu.sync_copy(
            hist_sh.at[pl.ds(s * slc, slc)],
            out_hbm.at[pl.ds(c * np_ + s * slc, slc)],
        )

    return deg_kernel


WIN = 16
CW = 128


def _sc_scatter(num_chunks: int, np_: int, dh: int):
    slc = np_ // NS
    grp = num_chunks // WIN
    mesh = plsc.VectorSubcoreMesh(core_axis_name="c", subcore_axis_name="s")

    @functools.partial(
        pl.kernel,
        out_type=jax.ShapeDtypeStruct((NC * np_, dh), jnp.float32),
        mesh=mesh,
        scratch_types=[
            pltpu.VMEM((WIN, CW), jnp.int32),
            pltpu.VMEM((WIN, CW), jnp.int32),
            [pltpu.VMEM((CW, dh), jnp.float32) for _ in range(2)],
            [pltpu.SemaphoreType.DMA for _ in range(4)],
            pltpu.VMEM_SHARED((np_, dh), jnp.float32),
        ],
    )
    def scat_kernel(hs_hbm, sidx_hbm, didx_hbm, out_hbm,
                    si_v, di_v, bufs, sems, acc_sh):
        c = lax.axis_index("c")
        s = lax.axis_index("s")
        w = c * NS + s
        sg = sems[:2]
        ss = sems[2:]
        zv = jnp.zeros((LN,), jnp.float32)

        @pl.loop(0, CW)
        def _(r):
            @pl.loop(0, dh, step=LN)
            def _(k):
                bufs[0][r, pl.ds(k, LN)] = zv

        @pl.loop(0, slc // CW)
        def _(k):
            pltpu.sync_copy(bufs[0], acc_sh.at[pl.ds(s * slc + k * CW, CW)])

        plsc.subcore_barrier()

        def gather(j, i):
            pltpu.async_copy(hs_hbm.at[si_v.at[j]], bufs[i], sg[i])

        def gather_wait(j, i):
            pltpu.make_async_copy(hs_hbm.at[si_v.at[j]], bufs[i],
                                  sg[i]).wait()

        def scat(j, i):
            pltpu.async_copy(bufs[i], acc_sh.at[di_v.at[j]], ss[i], add=True)

        def scat_wait(j, i):
            pltpu.make_async_copy(bufs[i], acc_sh.at[di_v.at[j]],
                                  ss[i]).wait()

        @pl.loop(0, grp)
        def _(g):
            pltpu.sync_copy(sidx_hbm.at[w * grp + g], si_v)
            pltpu.sync_copy(didx_hbm.at[s * grp + g], di_v)
            gather(0, 0)

            @pl.loop(0, WIN, step=2)
            def _(j):
                @pl.when(j > 0)
                def _():
                    scat_wait(j - 1, 1)

                gather(j + 1, 1)
                gather_wait(j, 0)
                scat(j, 0)

                @pl.when(j + 2 < WIN)
                def _():
                    scat_wait(j, 0)
                    gather(j + 2, 0)

                gather_wait(j + 1, 1)
                scat(j + 1, 1)

            scat_wait(WIN - 2, 0)
            scat_wait(WIN - 1, 1)

        plsc.subcore_barrier()
        pltpu.sync_copy(
            acc_sh.at[pl.ds(s * slc, slc)],
            out_hbm.at[pl.ds(c * np_ + s * slc, slc)],
        )

    return scat_kernel


def _tc_matmul(np_: int, blk: int, din: int, dh: int):
    nb = np_ // blk

    def body(x_ref, w_ref, da_ref, db_ref, o_ref):
        deg = da_ref[:, 0] + db_ref[:, 0] + 1.0
        dinv = lax.rsqrt(deg)
        xs = x_ref[...] * dinv[:, None]
        o_ref[...] = jnp.dot(xs, w_ref[...],
                             preferred_element_type=jnp.float32)

    return pl.pallas_call(
        body,
        grid=(nb, 2),
        in_specs=[
            pl.BlockSpec((blk, din), lambda i, j: (i, 0)),
            pl.BlockSpec((din, dh), lambda i, j: (0, j)),
            pl.BlockSpec((blk, DW), lambda i, j: (i, 0)),
            pl.BlockSpec((blk, DW), lambda i, j: (nb + i, 0)),
        ],
        out_specs=pl.BlockSpec((blk, dh), lambda i, j: (j * nb + i, 0)),
        out_shape=jax.ShapeDtypeStruct((NC * np_, dh), jnp.float32),
    )


def _tc_final(np_: int, blk: int, dh: int):
    nb = np_ // blk

    def body(aa_ref, ab_ref, ha_ref, hb_ref, da_ref, db_ref, b_ref, o_ref):
        deg = da_ref[:, 0] + db_ref[:, 0] + 1.0
        dinv = lax.rsqrt(deg)[:, None]
        left = dinv * (aa_ref[...] + ha_ref[...])
        right = dinv * (ab_ref[...] + hb_ref[...])
        o_ref[...] = jnp.maximum(
            jnp.concatenate([left, right], axis=1) + b_ref[...], 0.0)

    return pl.pallas_call(
        body,
        grid=(nb,),
        in_specs=[
            pl.BlockSpec((blk, dh), lambda i: (i, 0)),
            pl.BlockSpec((blk, dh), lambda i: (nb + i, 0)),
            pl.BlockSpec((blk, dh), lambda i: (i, 0)),
            pl.BlockSpec((blk, dh), lambda i: (nb + i, 0)),
            pl.BlockSpec((blk, DW), lambda i: (i, 0)),
            pl.BlockSpec((blk, DW), lambda i: (nb + i, 0)),
            pl.BlockSpec((1, 2 * dh), lambda i: (0, 0)),
        ],
        out_specs=pl.BlockSpec((blk, 2 * dh), lambda i: (i, 0)),
        out_shape=jax.ShapeDtypeStruct((np_, 2 * dh), jnp.float32),
    )


def kernel(x, spatial_edge_index, W, b):
    n, din = x.shape
    dout = W.shape[1]
    dh = dout // 2
    e = spatial_edge_index.shape[1]

    blk = 1024
    tile_q = NS * CW * WIN
    np_ = ((n + 1 + 2047) // 2048) * 2048
    np_ = ((np_ + blk - 1) // blk) * blk
    ep = ((e + tile_q - 1) // tile_q) * tile_q

    src = spatial_edge_index[0].astype(jnp.int32)
    dst = spatial_edge_index[1].astype(jnp.int32)
    pad = ep - e
    srcp = jnp.concatenate([src, jnp.full((pad,), n, jnp.int32)])
    dstp = jnp.concatenate([dst, jnp.full((pad,), n, jnp.int32)])

    ch_m = ep // (NS * CW)
    ch_d = ep // (NC * NS * CHW)
    grp = ch_m // WIN

    src_by_s = srcp.reshape(NS, ch_m, CW)
    src_arr = jnp.concatenate([src_by_s, src_by_s + np_],
                              axis=0).reshape(NC * NS * grp, WIN, CW)
    dst_arr = dstp.reshape(NS * grp, WIN, CW)
    deg_idx = dstp.reshape(NC * NS, ch_d, CHW)

    x_pad = jnp.concatenate(
        [x, jnp.zeros((np_ - n, din), x.dtype)], axis=0)

    deg01 = _sc_degree(ch_d, np_)(deg_idx)
    hs = _tc_matmul(np_, blk, din, dh)(x_pad, W, deg01, deg01)
    acc = _sc_scatter(ch_m, np_, dh)(hs, src_arr, dst_arr)
    out_p = _tc_final(np_, blk, dh)(acc, acc, hs, hs, deg01, deg01,
                                    b.reshape(1, dout))
    return out_p[:n]
